# baseline (speedup 1.0000x reference)
import functools
import jax
import jax.numpy as jnp
from jax import lax
from jax.experimental import pallas as pl
from jax.experimental.pallas import tpu as pltpu
from jax.experimental.pallas import tpu_sc as plsc

_B = 4096
_T = 26
_V = 100000
_D = 32
_DF = 128
_H = 512

_NC = 2
_NS = 16
_NW = _NC * _NS


def _sc_gather_body(tab_hbm, idx_hbm, out_hbm, buf_v, idx_v, out_v, semt, semi, semo):
    c = lax.axis_index("c")
    s = lax.axis_index("s")
    w = s * _NC + c

    def idx_slot(t):
        return idx_v.at[pl.ds((t % 2) * _B, _B)]

    pltpu.make_async_copy(tab_hbm.at[0, w], buf_v, semt).start()
    pltpu.make_async_copy(idx_hbm.at[0], idx_slot(0), semi).start()

    def per_table(t, _):
        iv = idx_slot(t)
        pltpu.make_async_copy(idx_hbm.at[t], iv, semi).wait()

        @pl.when(t + 1 < _T)
        def _():
            pltpu.make_async_copy(idx_hbm.at[t + 1], idx_slot(t + 1), semi).start()

        pltpu.make_async_copy(tab_hbm.at[t, w], buf_v, semt).wait()

        @pl.when(t >= 1)
        def _():
            pltpu.make_async_copy(
                out_v, out_hbm.at[(t - 1) * _D + w], semo).wait()

        def gather4(k, _):
            for u in range(4):
                sl = pl.ds(k * 64 + u * 16, 16)
                out_v[sl] = plsc.load_gather(buf_v, [iv[sl]])
            return 0

        lax.fori_loop(0, _B // 64, gather4, 0)

        @pl.when(t + 1 < _T)
        def _():
            pltpu.make_async_copy(tab_hbm.at[t + 1, w], buf_v, semt).start()

        pltpu.make_async_copy(out_v, out_hbm.at[t * _D + w], semo).start()
        return 0

    lax.fori_loop(0, _T, per_table, 0)
    pltpu.make_async_copy(
        out_v, out_hbm.at[(_T - 1) * _D + w], semo).wait()


@jax.jit
def _sc_gather(tab, idx_t):
    mesh = plsc.VectorSubcoreMesh(core_axis_name="c", subcore_axis_name="s")
    return pl.kernel(
        _sc_gather_body,
        out_type=jax.ShapeDtypeStruct((_T * _D, _B), jnp.float32),
        mesh=mesh,
        scratch_types=[
            pltpu.VMEM((_V,), jnp.float32),
            pltpu.VMEM((2 * _B,), jnp.int32),
            pltpu.VMEM((_B,), jnp.float32),
            pltpu.SemaphoreType.DMA,
            pltpu.SemaphoreType.DMA,
            pltpu.SemaphoreType.DMA,
        ],
        compiler_params=pltpu.CompilerParams(
            use_tc_tiling_on_sc=True, needs_layout_passes=False),
    )(tab, idx_t)


def _mlp_body(df_ref, embt_ref, w1_ref, b1_ref, w2_ref, b2_ref, w3_ref, b3_ref,
              w4_ref, b4_ref, out_ref):
    f32 = jnp.float32
    h = jnp.maximum(
        jnp.dot(df_ref[...], w1_ref[...], preferred_element_type=f32) + b1_ref[...], 0.0)
    dx = jnp.maximum(
        jnp.dot(h, w2_ref[...], preferred_element_type=f32) + b2_ref[...], 0.0)
    emb_w3 = lax.dot_general(
        embt_ref[...], w3_ref[_D:, :],
        dimension_numbers=(((0,), (0,)), ((), ())),
        preferred_element_type=f32)
    g = (jnp.dot(dx, w3_ref[0:_D, :], preferred_element_type=f32)
         + emb_w3 + b3_ref[...])
    g = jnp.maximum(g, 0.0)
    out_ref[...] = jnp.maximum(
        jnp.dot(g, w4_ref[...], preferred_element_type=f32) + b4_ref[...], 0.0)


@functools.partial(jax.jit, static_argnames=("bb",))
def _tc_mlp(df, embt, w1, b1, w2, b2, w3, b3, w4, b4, bb=512):
    grid = (_B // bb,)
    full = lambda shape: pl.BlockSpec(shape, lambda i: (0, 0))
    return pl.pallas_call(
        _mlp_body,
        grid=grid,
        in_specs=[
            pl.BlockSpec((bb, _DF), lambda i: (i, 0)),
            pl.BlockSpec((_T * _D, bb), lambda i: (0, i)),
            full((_DF, _DF)),
            full((1, _DF)),
            full((_DF, _D)),
            full((1, _D)),
            full((_D + _T * _D, _H)),
            full((1, _H)),
            full((_H, 1)),
            full((1, 1)),
        ],
        out_specs=pl.BlockSpec((bb, 1), lambda i: (i, 0)),
        out_shape=jax.ShapeDtypeStruct((_B, 1), jnp.float32),
        compiler_params=pltpu.CompilerParams(
            dimension_semantics=("arbitrary",),
        ),
    )(df, embt, w1, b1, w2, b2, w3, b3, w4, b4)


def kernel(dense_features, sharded_sparse_features, tables, w1, b1, w2, b2, w3, b3, w4, b4):
    tt = jnp.transpose(tables, (0, 2, 1))
    idx_t = sharded_sparse_features.astype(jnp.int32).T
    embt = _sc_gather(tt, idx_t)
    return _tc_mlp(
        dense_features, embt,
        w1, b1.reshape(1, _DF),
        w2, b2.reshape(1, _D),
        w3, b3.reshape(1, _H),
        w4, b4.reshape(1, 1),
    )


# speedup vs baseline: 1.1118x; 1.1118x over previous
import functools
import jax
import jax.numpy as jnp
from jax import lax
from jax.experimental import pallas as pl
from jax.experimental.pallas import tpu as pltpu
from jax.experimental.pallas import tpu_sc as plsc

_B = 4096
_T = 26
_V = 100000
_D = 32
_DF = 128
_H = 512

_NC = 2
_NS = 16
_NW = _NC * _NS


def _sc_gather_body(tab_hbm, idx_hbm, out_hbm, buf_v, idx_v, out_v, semt, semi, semo):
    c = lax.axis_index("c")
    s = lax.axis_index("s")
    w = s * _NC + c

    def per_table(t, _):
        cpt = pltpu.make_async_copy(tab_hbm.at[t, w], buf_v, semt)
        cpt.start()
        cpi = pltpu.make_async_copy(idx_hbm.at[t], idx_v, semi)
        cpi.start()
        cpi.wait()
        cpt.wait()

        def gather4(k, _):
            for u in range(4):
                sl = pl.ds(k * 64 + u * 16, 16)
                out_v[sl] = plsc.load_gather(buf_v, [idx_v[sl]])
            return 0

        lax.fori_loop(0, _B // 64, gather4, 0)
        pltpu.sync_copy(out_v, out_hbm.at[t * _D + w])
        return 0

    lax.fori_loop(0, _T, per_table, 0)


@jax.jit
def _sc_gather(tab, idx_t):
    mesh = plsc.VectorSubcoreMesh(core_axis_name="c", subcore_axis_name="s")
    return pl.kernel(
        _sc_gather_body,
        out_type=jax.ShapeDtypeStruct((_T * _D, _B), jnp.float32),
        mesh=mesh,
        scratch_types=[
            pltpu.VMEM((_V,), jnp.float32),
            pltpu.VMEM((_B,), jnp.int32),
            pltpu.VMEM((_B,), jnp.float32),
            pltpu.SemaphoreType.DMA,
            pltpu.SemaphoreType.DMA,
            pltpu.SemaphoreType.DMA,
        ],
        compiler_params=pltpu.CompilerParams(
            use_tc_tiling_on_sc=True, needs_layout_passes=False),
    )(tab, idx_t)


def _mlp_body(df_ref, embt_ref, w1_ref, b1_ref, w2_ref, b2_ref, w3_ref, b3_ref,
              w4_ref, b4_ref, out_ref):
    f32 = jnp.float32
    h = jnp.maximum(
        jnp.dot(df_ref[...], w1_ref[...], preferred_element_type=f32) + b1_ref[...], 0.0)
    dx = jnp.maximum(
        jnp.dot(h, w2_ref[...], preferred_element_type=f32) + b2_ref[...], 0.0)
    emb_w3 = lax.dot_general(
        embt_ref[...], w3_ref[_D:, :],
        dimension_numbers=(((0,), (0,)), ((), ())),
        preferred_element_type=f32)
    g = (jnp.dot(dx, w3_ref[0:_D, :], preferred_element_type=f32)
         + emb_w3 + b3_ref[...])
    g = jnp.maximum(g, 0.0)
    out_ref[...] = jnp.maximum(
        jnp.dot(g, w4_ref[...], preferred_element_type=f32) + b4_ref[...], 0.0)


@functools.partial(jax.jit, static_argnames=("bb",))
def _tc_mlp(df, embt, w1, b1, w2, b2, w3, b3, w4, b4, bb=512):
    grid = (_B // bb,)
    full = lambda shape: pl.BlockSpec(shape, lambda i: (0, 0))
    return pl.pallas_call(
        _mlp_body,
        grid=grid,
        in_specs=[
            pl.BlockSpec((bb, _DF), lambda i: (i, 0)),
            pl.BlockSpec((_T * _D, bb), lambda i: (0, i)),
            full((_DF, _DF)),
            full((1, _DF)),
            full((_DF, _D)),
            full((1, _D)),
            full((_D + _T * _D, _H)),
            full((1, _H)),
            full((_H, 1)),
            full((1, 1)),
        ],
        out_specs=pl.BlockSpec((bb, 1), lambda i: (i, 0)),
        out_shape=jax.ShapeDtypeStruct((_B, 1), jnp.float32),
        compiler_params=pltpu.CompilerParams(
            dimension_semantics=("arbitrary",),
        ),
    )(df, embt, w1, b1, w2, b2, w3, b3, w4, b4)


def kernel(dense_features, sharded_sparse_features, tables, w1, b1, w2, b2, w3, b3, w4, b4):
    tt = jnp.transpose(tables, (0, 2, 1))
    idx_t = sharded_sparse_features.astype(jnp.int32).T
    embt = _sc_gather(tt, idx_t)
    return _tc_mlp(
        dense_features, embt,
        w1, b1.reshape(1, _DF),
        w2, b2.reshape(1, _D),
        w3, b3.reshape(1, _H),
        w4, b4.reshape(1, 1),
    )


# speedup vs baseline: 1.1130x; 1.0010x over previous
import functools
import jax
import jax.numpy as jnp
from jax import lax
from jax.experimental import pallas as pl
from jax.experimental.pallas import tpu as pltpu
from jax.experimental.pallas import tpu_sc as plsc

_B = 4096
_T = 26
_V = 100000
_D = 32
_DF = 128
_H = 512

_NC = 2
_NS = 16
_NW = _NC * _NS


def _sc_gather_body(tab_hbm, idx_hbm, out_hbm, buf_v, idx_v, out_v, semt, semi, semo):
    c = lax.axis_index("c")
    s = lax.axis_index("s")
    w = s * _NC + c

    def per_table(t, _):
        cpt = pltpu.make_async_copy(tab_hbm.at[t, w], buf_v, semt)
        cpt.start()
        cpi = pltpu.make_async_copy(idx_hbm.at[t], idx_v, semi)
        cpi.start()
        cpi.wait()
        cpt.wait()

        def gather8(k, _):
            for u in range(8):
                sl = pl.ds(k * 128 + u * 16, 16)
                out_v[sl] = plsc.load_gather(buf_v, [idx_v[sl]])
            return 0

        lax.fori_loop(0, _B // 128, gather8, 0)
        pltpu.sync_copy(out_v, out_hbm.at[t * _D + w])
        return 0

    lax.fori_loop(0, _T, per_table, 0)


@jax.jit
def _sc_gather(tab, idx_t):
    mesh = plsc.VectorSubcoreMesh(core_axis_name="c", subcore_axis_name="s")
    return pl.kernel(
        _sc_gather_body,
        out_type=jax.ShapeDtypeStruct((_T * _D, _B), jnp.float32),
        mesh=mesh,
        scratch_types=[
            pltpu.VMEM((_V,), jnp.float32),
            pltpu.VMEM((_B,), jnp.int32),
            pltpu.VMEM((_B,), jnp.float32),
            pltpu.SemaphoreType.DMA,
            pltpu.SemaphoreType.DMA,
            pltpu.SemaphoreType.DMA,
        ],
        compiler_params=pltpu.CompilerParams(
            use_tc_tiling_on_sc=True, needs_layout_passes=False),
    )(tab, idx_t)


def _mlp_body(df_ref, embt_ref, w1_ref, b1_ref, w2_ref, b2_ref, w3_ref, b3_ref,
              w4_ref, b4_ref, out_ref):
    f32 = jnp.float32
    h = jnp.maximum(
        jnp.dot(df_ref[...], w1_ref[...], preferred_element_type=f32) + b1_ref[...], 0.0)
    dx = jnp.maximum(
        jnp.dot(h, w2_ref[...], preferred_element_type=f32) + b2_ref[...], 0.0)
    emb_w3 = lax.dot_general(
        embt_ref[...], w3_ref[_D:, :],
        dimension_numbers=(((0,), (0,)), ((), ())),
        preferred_element_type=f32)
    g = (jnp.dot(dx, w3_ref[0:_D, :], preferred_element_type=f32)
         + emb_w3 + b3_ref[...])
    g = jnp.maximum(g, 0.0)
    out_ref[...] = jnp.maximum(
        jnp.dot(g, w4_ref[...], preferred_element_type=f32) + b4_ref[...], 0.0)


@functools.partial(jax.jit, static_argnames=("bb",))
def _tc_mlp(df, embt, w1, b1, w2, b2, w3, b3, w4, b4, bb=512):
    grid = (_B // bb,)
    full = lambda shape: pl.BlockSpec(shape, lambda i: (0, 0))
    return pl.pallas_call(
        _mlp_body,
        grid=grid,
        in_specs=[
            pl.BlockSpec((bb, _DF), lambda i: (i, 0)),
            pl.BlockSpec((_T * _D, bb), lambda i: (0, i)),
            full((_DF, _DF)),
            full((1, _DF)),
            full((_DF, _D)),
            full((1, _D)),
            full((_D + _T * _D, _H)),
            full((1, _H)),
            full((_H, 1)),
            full((1, 1)),
        ],
        out_specs=pl.BlockSpec((bb, 1), lambda i: (i, 0)),
        out_shape=jax.ShapeDtypeStruct((_B, 1), jnp.float32),
        compiler_params=pltpu.CompilerParams(
            dimension_semantics=("arbitrary",),
        ),
    )(df, embt, w1, b1, w2, b2, w3, b3, w4, b4)


def kernel(dense_features, sharded_sparse_features, tables, w1, b1, w2, b2, w3, b3, w4, b4):
    tt = jnp.transpose(tables, (0, 2, 1))
    idx_t = sharded_sparse_features.astype(jnp.int32).T
    embt = _sc_gather(tt, idx_t)
    return _tc_mlp(
        dense_features, embt,
        w1, b1.reshape(1, _DF),
        w2, b2.reshape(1, _D),
        w3, b3.reshape(1, _H),
        w4, b4.reshape(1, 1),
    )


# speedup vs baseline: 1.1282x; 1.0137x over previous
import functools
import jax
import jax.numpy as jnp
from jax import lax
from jax.experimental import pallas as pl
from jax.experimental.pallas import tpu as pltpu
from jax.experimental.pallas import tpu_sc as plsc

_B = 4096
_T = 26
_V = 100000
_D = 32
_DF = 128
_H = 512

_NC = 2
_NS = 16
_NW = _NC * _NS


def _sc_gather_body(tab_hbm, idx_hbm, out_hbm, buf_v, idx_v, out_v, semt, semi, semo):
    c = lax.axis_index("c")
    s = lax.axis_index("s")
    w = s * _NC + c

    def per_table(t, _):
        cpt = pltpu.make_async_copy(tab_hbm.at[t, w], buf_v, semt)
        cpt.start()
        cpi = pltpu.make_async_copy(idx_hbm.at[t], idx_v, semi)
        cpi.start()
        cpi.wait()
        cpt.wait()

        def gather8(k, _):
            for u in range(8):
                sl = pl.ds(k * 128 + u * 16, 16)
                out_v[sl] = plsc.load_gather(buf_v, [idx_v[sl]])
            return 0

        lax.fori_loop(0, _B // 128, gather8, 0)
        pltpu.sync_copy(out_v, out_hbm.at[t * _D + w])
        return 0

    lax.fori_loop(0, _T, per_table, 0)


@jax.jit
def _sc_gather(tab, idx_t):
    mesh = plsc.VectorSubcoreMesh(core_axis_name="c", subcore_axis_name="s")
    return pl.kernel(
        _sc_gather_body,
        out_type=jax.ShapeDtypeStruct((_T * _D, _B), jnp.float32),
        mesh=mesh,
        scratch_types=[
            pltpu.VMEM((_V,), jnp.float32),
            pltpu.VMEM((_B,), jnp.int32),
            pltpu.VMEM((_B,), jnp.float32),
            pltpu.SemaphoreType.DMA,
            pltpu.SemaphoreType.DMA,
            pltpu.SemaphoreType.DMA,
        ],
        compiler_params=pltpu.CompilerParams(
            use_tc_tiling_on_sc=True, needs_layout_passes=False),
    )(tab, idx_t)


def _mlp_body(df_ref, embt_ref, w1_ref, b1_ref, w2_ref, b2_ref, w3_ref, b3_ref,
              w4_ref, b4_ref, out_ref):
    f32 = jnp.float32
    h = jnp.maximum(
        jnp.dot(df_ref[...], w1_ref[...], preferred_element_type=f32) + b1_ref[...], 0.0)
    dx = jnp.maximum(
        jnp.dot(h, w2_ref[...], preferred_element_type=f32) + b2_ref[...], 0.0)
    emb_w3 = lax.dot_general(
        embt_ref[...], w3_ref[_D:, :],
        dimension_numbers=(((0,), (0,)), ((), ())),
        preferred_element_type=f32)
    g = (jnp.dot(dx, w3_ref[0:_D, :], preferred_element_type=f32)
         + emb_w3 + b3_ref[...])
    g = jnp.maximum(g, 0.0)
    out_ref[...] = jnp.maximum(
        jnp.dot(g, w4_ref[...], preferred_element_type=f32) + b4_ref[...], 0.0)


@functools.partial(jax.jit, static_argnames=("bb",))
def _tc_mlp(df, embt, w1, b1, w2, b2, w3, b3, w4, b4, bb=1024):
    grid = (_B // bb,)
    full = lambda shape: pl.BlockSpec(shape, lambda i: (0, 0))
    return pl.pallas_call(
        _mlp_body,
        grid=grid,
        in_specs=[
            pl.BlockSpec((bb, _DF), lambda i: (i, 0)),
            pl.BlockSpec((_T * _D, bb), lambda i: (0, i)),
            full((_DF, _DF)),
            full((1, _DF)),
            full((_DF, _D)),
            full((1, _D)),
            full((_D + _T * _D, _H)),
            full((1, _H)),
            full((_H, 1)),
            full((1, 1)),
        ],
        out_specs=pl.BlockSpec((bb, 1), lambda i: (i, 0)),
        out_shape=jax.ShapeDtypeStruct((_B, 1), jnp.float32),
        compiler_params=pltpu.CompilerParams(
            dimension_semantics=("arbitrary",),
        ),
    )(df, embt, w1, b1, w2, b2, w3, b3, w4, b4)


def kernel(dense_features, sharded_sparse_features, tables, w1, b1, w2, b2, w3, b3, w4, b4):
    tt = jnp.transpose(tables, (0, 2, 1))
    idx_t = sharded_sparse_features.astype(jnp.int32).T
    embt = _sc_gather(tt, idx_t)
    return _tc_mlp(
        dense_features, embt,
        w1, b1.reshape(1, _DF),
        w2, b2.reshape(1, _D),
        w3, b3.reshape(1, _H),
        w4, b4.reshape(1, 1),
    )
